# baseline (speedup 1.0000x reference)
import functools

import jax
import jax.numpy as jnp
import numpy as np
from jax import lax
from jax.experimental import pallas as pl
from jax.experimental.pallas import tpu as pltpu

_EMB = 32
_LEN = 512
_LB = -15.0
_UB = 16.0
_LN_BASE = float(np.log(256.0 ** (1.0 / 15.0)))

_BJ = 32


def _bucket(d):
    sign = jnp.sign(d).astype(jnp.float32)
    a = jnp.abs(d).astype(jnp.float32)
    v = jnp.floor(jnp.log(a) / _LN_BASE + 1.0)
    v = jnp.where(v < 0, 0.0, v)
    v = v * sign
    v = jnp.where(v < _LB, _LB, v)
    v = jnp.where(v > _UB, _UB, v)
    return (v - _LB).astype(jnp.int32)


def _plane_body(o_ref, lohi_ref):
    jb = pl.program_id(0)

    @pl.when(jb == 0)
    def _():
        dd = lax.broadcasted_iota(jnp.int32, (_EMB, 1024), 1) - (_LEN - 1)
        e = lax.broadcasted_iota(jnp.int32, (_EMB, 1024), 0)
        m = _bucket(dd) == e
        dfl = dd.astype(jnp.float32)
        lohi_ref[:, 0:1] = jnp.min(jnp.where(m, dfl, 1e9), axis=1, keepdims=True)
        lohi_ref[:, 1:2] = jnp.max(jnp.where(m, dfl, -1e9), axis=1, keepdims=True)

    lo = lohi_ref[:, 0:1].reshape(1, 1, _EMB, 1)
    hi = lohi_ref[:, 1:2].reshape(1, 1, _EMB, 1)
    kk = lax.broadcasted_iota(jnp.int32, (1, _BJ, _EMB, _LEN), 3)
    jj = lax.broadcasted_iota(jnp.int32, (1, _BJ, _EMB, _LEN), 1)
    d = (kk - jj - jb * _BJ).astype(jnp.float32)
    v = jnp.clip(jnp.minimum(d - lo + 1.0, hi - d + 1.0), 0.0, 1.0)
    o_ref[...] = jnp.broadcast_to(v, (4, _BJ, _EMB, _LEN))


_planes = pl.pallas_call(
    _plane_body,
    grid=(_LEN // _BJ,),
    out_specs=pl.BlockSpec((4, _BJ, _EMB, _LEN), lambda j: (0, j, 0, 0)),
    out_shape=jax.ShapeDtypeStruct((4, _LEN, _EMB, _LEN), jnp.float32),
    scratch_shapes=[pltpu.VMEM((_EMB, 128), jnp.float32)],
)


def kernel(x):
    batch, length = x.shape
    out = _planes()
    return jnp.transpose(out, (0, 1, 3, 2))
